# Initial kernel scaffold; baseline (speedup 1.0000x reference)
import jax
import jax.numpy as jnp
from jax.experimental import pallas as pl


def kernel(x_nchw, weight_oihw, beta):
    raise NotImplementedError("write your pallas kernel here")


# speedup vs baseline: 1.2431x; 1.2431x over previous
import functools

import numpy as np

import jax
import jax.numpy as jnp
from jax import lax
from jax.experimental import pallas as pl
from jax.experimental.pallas import tpu as pltpu


def _tap_geometry(H, W):
    HW = H * W
    hh = np.arange(H)[:, None]
    ww = np.arange(W)[None, :]
    masks = np.ones((9, HW), np.float32)
    plan = []
    for kh in range(3):
        for kw in range(3):
            t = kh * 3 + kw
            dh, dw = kh - 1, kw - 1
            shift = int((-(dh * W + dw)) % HW)
            valid = ((hh + dh >= 0) & (hh + dh < H) &
                     (ww + dw >= 0) & (ww + dw < W))
            masks[t] = valid.reshape(HW).astype(np.float32)
            plan.append((t, shift, not (dh == 0 and dw == 0)))
    return tuple(plan), masks


def _conv_image(x, w_ref, m_ref, plan):
    acc = None
    for t, shift, masked in plan:
        tap = x if shift == 0 else pltpu.roll(x, shift, axis=1)
        if masked:
            tap = tap * m_ref[t:t + 1, :]
        part = jnp.dot(w_ref[t], tap, preferred_element_type=jnp.float32)
        acc = part if acc is None else acc + part
    return acc


def _conv_stats_kernel(x_ref, w_ref, m_ref, y_ref, s_ref, *, plan):
    acc = _conv_image(x_ref[0], w_ref, m_ref, plan)
    y_ref[...] = acc.astype(jnp.bfloat16)[None]
    s_ref[0, 0, :] = jnp.sum(acc, axis=1)
    s_ref[0, 1, :] = jnp.sum(acc * acc, axis=1)


def _norm_relu_kernel(y_ref, scale_ref, shift_ref, o_ref):
    y = y_ref[0].astype(jnp.float32)
    out = jnp.maximum(y * scale_ref[...] + shift_ref[...], 0.0)
    o_ref[...] = out[None]


@jax.jit
def _conv_block(x_nchw, weight_oihw, beta):
    eps = 1e-5
    N, Cin, H, W = x_nchw.shape
    Cout = weight_oihw.shape[0]
    HW = H * W

    plan, masks_np = _tap_geometry(H, W)

    x = x_nchw.reshape(N, Cin, HW).astype(jnp.bfloat16)
    w_taps32 = jnp.transpose(weight_oihw.astype(jnp.float32),
                             (2, 3, 0, 1)).reshape(9, Cout, Cin)
    w_taps = w_taps32.astype(jnp.bfloat16)
    masks = jnp.asarray(masks_np, dtype=jnp.bfloat16)

    conv_flops = 2 * Cout * 9 * Cin * HW

    y_raw, stats = pl.pallas_call(
        functools.partial(_conv_stats_kernel, plan=plan),
        out_shape=(
            jax.ShapeDtypeStruct((N, Cout, HW), jnp.bfloat16),
            jax.ShapeDtypeStruct((N, 2, Cout), jnp.float32),
        ),
        grid=(N,),
        in_specs=[
            pl.BlockSpec((1, Cin, HW), lambda n: (n, 0, 0)),
            pl.BlockSpec((9, Cout, Cin), lambda n: (0, 0, 0)),
            pl.BlockSpec((9, HW), lambda n: (0, 0)),
        ],
        out_specs=(
            pl.BlockSpec((1, Cout, HW), lambda n: (n, 0, 0)),
            pl.BlockSpec((1, 2, Cout), lambda n: (n, 0, 0)),
        ),
        compiler_params=pltpu.CompilerParams(
            dimension_semantics=("parallel",)),
        cost_estimate=pl.CostEstimate(
            flops=N * conv_flops,
            transcendentals=0,
            bytes_accessed=2 * (N * Cin * HW + 9 * Cout * Cin + 9 * HW
                                + N * Cout * HW) + 4 * N * 2 * Cout),
    )(x, w_taps, masks)

    count = float(N * HW)
    tot = jnp.sum(stats, axis=0)
    mean = tot[0] / count
    var = jnp.maximum(tot[1] / count - mean * mean, 0.0)
    scale = lax.rsqrt(var + eps)
    shift = beta.astype(jnp.float32) - mean * scale

    y = pl.pallas_call(
        _norm_relu_kernel,
        out_shape=jax.ShapeDtypeStruct((N, Cout, HW), jnp.float32),
        grid=(N,),
        in_specs=[
            pl.BlockSpec((1, Cout, HW), lambda n: (n, 0, 0)),
            pl.BlockSpec((Cout, 1), lambda n: (0, 0)),
            pl.BlockSpec((Cout, 1), lambda n: (0, 0)),
        ],
        out_specs=pl.BlockSpec((1, Cout, HW), lambda n: (n, 0, 0)),
        compiler_params=pltpu.CompilerParams(
            dimension_semantics=("parallel",)),
        cost_estimate=pl.CostEstimate(
            flops=2 * N * Cout * HW,
            transcendentals=0,
            bytes_accessed=2 * N * Cout * HW + 4 * N * Cout * HW
                           + 8 * Cout),
    )(y_raw, scale.reshape(Cout, 1), shift.reshape(Cout, 1))

    return y.reshape(N, Cout, H, W)


def kernel(x_nchw, weight_oihw, beta):
    return _conv_block(x_nchw, weight_oihw, beta)


# speedup vs baseline: 1.3337x; 1.0729x over previous
import functools

import numpy as np

import jax
import jax.numpy as jnp
from jax import lax
from jax.experimental import pallas as pl
from jax.experimental.pallas import tpu as pltpu


def _col_masks(H, W):
    HW = H * W
    col = np.arange(HW) % W
    m = np.ones((2, HW), np.float32)
    m[0, col == W - 1] = 0.0
    m[1, col == 0] = 0.0
    return m


def _row_masks(H, W):
    HW = H * W
    m = np.ones((2, HW), np.float32)
    m[0, :W] = 0.0
    m[1, HW - W:] = 0.0
    return m


def _conv_stats_kernel(x_ref, w_ref, cm_ref, rm_ref, o_ref, s_ref, *, H, W):
    HW = H * W
    x = x_ref[0].astype(jnp.bfloat16)
    xl = x * cm_ref[0:1, :]
    xr = x * cm_ref[1:2, :]

    def group(dh):
        part = None
        for dw in (-1, 0, 1):
            src = (xl, x, xr)[dw + 1]
            shift = (-(dh * W + dw)) % HW
            tap = src if shift == 0 else pltpu.roll(src, shift, axis=1)
            t = (dh + 1) * 3 + (dw + 1)
            d = jnp.dot(w_ref[t], tap, preferred_element_type=jnp.float32)
            part = d if part is None else part + d
        return part

    acc = group(-1) * rm_ref[0:1, :]
    acc = acc + group(0)
    acc = acc + group(1) * rm_ref[1:2, :]

    o_ref[...] = acc.astype(jnp.bfloat16)[None]
    s_ref[0, :, 0:1] = jnp.sum(acc, axis=1, keepdims=True)
    s_ref[0, :, 1:2] = jnp.sum(acc * acc, axis=1, keepdims=True)


def _norm_relu_kernel(s_ref, beta_ref, y_ref, o_ref, *, inv_count, eps):
    tot = jnp.sum(s_ref[...], axis=0)
    mean = tot[:, 0:1] * inv_count
    var = jnp.maximum(tot[:, 1:2] * inv_count - mean * mean, 0.0)
    scale = lax.rsqrt(var + eps)
    shift = beta_ref[...] - mean * scale
    y = y_ref[0].astype(jnp.float32)
    o_ref[...] = jnp.maximum(y * scale + shift, 0.0)[None]


@jax.jit
def _conv_block(x_nchw, weight_oihw, beta):
    eps = 1e-5
    N, Cin, H, W = x_nchw.shape
    Cout = weight_oihw.shape[0]
    HW = H * W

    x = x_nchw.reshape(N, Cin, HW)
    w_taps = jnp.transpose(weight_oihw.astype(jnp.float32),
                           (2, 3, 0, 1)).reshape(9, Cout, Cin)
    w_taps = w_taps.astype(jnp.bfloat16)
    cmasks = jnp.asarray(_col_masks(H, W), dtype=jnp.bfloat16)
    rmasks = jnp.asarray(_row_masks(H, W), dtype=jnp.float32)

    conv_flops = 2 * Cout * 9 * Cin * HW

    y_raw, stats = pl.pallas_call(
        functools.partial(_conv_stats_kernel, H=H, W=W),
        out_shape=(
            jax.ShapeDtypeStruct((N, Cout, HW), jnp.bfloat16),
            jax.ShapeDtypeStruct((N, Cout, 2), jnp.float32),
        ),
        grid=(N,),
        in_specs=[
            pl.BlockSpec((1, Cin, HW), lambda n: (n, 0, 0)),
            pl.BlockSpec((9, Cout, Cin), lambda n: (0, 0, 0)),
            pl.BlockSpec((2, HW), lambda n: (0, 0)),
            pl.BlockSpec((2, HW), lambda n: (0, 0)),
        ],
        out_specs=(
            pl.BlockSpec((1, Cout, HW), lambda n: (n, 0, 0)),
            pl.BlockSpec((1, Cout, 2), lambda n: (n, 0, 0)),
        ),
        compiler_params=pltpu.CompilerParams(
            dimension_semantics=("parallel",)),
        cost_estimate=pl.CostEstimate(
            flops=N * conv_flops,
            transcendentals=0,
            bytes_accessed=4 * N * Cin * HW + 2 * (9 * Cout * Cin + 2 * HW
                                + N * Cout * HW) + 4 * N * Cout * 2),
    )(x, w_taps, cmasks, rmasks)

    y = pl.pallas_call(
        functools.partial(_norm_relu_kernel,
                          inv_count=1.0 / float(N * HW), eps=eps),
        out_shape=jax.ShapeDtypeStruct((N, Cout, HW), jnp.float32),
        grid=(N,),
        in_specs=[
            pl.BlockSpec((N, Cout, 2), lambda n: (0, 0, 0)),
            pl.BlockSpec((Cout, 1), lambda n: (0, 0)),
            pl.BlockSpec((1, Cout, HW), lambda n: (n, 0, 0)),
        ],
        out_specs=pl.BlockSpec((1, Cout, HW), lambda n: (n, 0, 0)),
        compiler_params=pltpu.CompilerParams(
            dimension_semantics=("parallel",)),
        cost_estimate=pl.CostEstimate(
            flops=2 * N * Cout * HW,
            transcendentals=Cout,
            bytes_accessed=2 * N * Cout * HW + 4 * N * Cout * HW
                           + 4 * (N * Cout * 2 + Cout)),
    )(stats, beta.astype(jnp.float32).reshape(Cout, 1), y_raw)

    return y.reshape(N, Cout, H, W)


def kernel(x_nchw, weight_oihw, beta):
    return _conv_block(x_nchw, weight_oihw, beta)
